# baseline (speedup 1.0000x reference)
import functools

import jax
import jax.numpy as jnp
from jax import lax
from jax.experimental import pallas as pl
from jax.experimental.pallas import tpu as pltpu
from jax.experimental.pallas import tpu_sc as plsc

_WD = 0.01
_B = 16384
_D = 64
_NC = 2
_NS = 16
_NW = _NC * _NS
_BPW = _B // _NW
_CH = 128
_NCH = _BPW // _CH
_GPC = _CH // 16


def _sc_body(u_hbm, i_hbm, j_hbm, T_hbm, bias_hbm, x_out, reg_out,
             idx_u, idx_i, idx_j, ue_v, ie_v, je_v, bi_v, bj_v, xv, pu, pj,
             regv, sem_i, sem_a, sem_b):
    wid = lax.axis_index("s") * _NC + lax.axis_index("c")
    base = wid * _BPW

    stage = []
    for k in range(_NCH):
        src = pl.ds(base + k * _CH, _CH)
        stage.append(pltpu.async_copy(u_hbm.at[src], idx_u.at[k], sem_i))
        stage.append(pltpu.async_copy(i_hbm.at[src], idx_i.at[k], sem_i))
        stage.append(pltpu.async_copy(j_hbm.at[src], idx_j.at[k], sem_i))
    for c in stage:
        c.wait()

    sems = (sem_a, sem_b)

    def fire(k):
        p = k % 2
        sem = sems[p]
        sl = pl.ds(k * _CH, _CH)
        return [
            pltpu.async_copy(T_hbm.at[idx_u.at[k]], ue_v.at[p], sem),
            pltpu.async_copy(T_hbm.at[idx_i.at[k]], ie_v.at[p], sem),
            pltpu.async_copy(T_hbm.at[idx_j.at[k]], je_v.at[p], sem),
            pltpu.async_copy(bias_hbm.at[idx_i.at[k]], bi_v.at[sl], sem),
            pltpu.async_copy(bias_hbm.at[idx_j.at[k]], bj_v.at[sl], sem),
        ]

    lanes = lax.iota(jnp.int32, 16)
    lanes17 = lanes * 17
    zeros = jnp.zeros((16,), jnp.float32)

    def make_group(p, k):
        def group(g, reg_acc):
            acc_sq = zeros
            for e in range(16):
                row = g * 16 + e
                u0, u1 = plsc.unpack(
                    ue_v[p, row, pl.ds(0, 32)],
                    format=plsc.PackFormat.INTERLEAVED,
                )
                u2, u3 = plsc.unpack(
                    ue_v[p, row, pl.ds(32, 32)],
                    format=plsc.PackFormat.INTERLEAVED,
                )
                i0, i1 = plsc.unpack(
                    ie_v[p, row, pl.ds(64, 32)],
                    format=plsc.PackFormat.INTERLEAVED,
                )
                i2, i3 = plsc.unpack(
                    ie_v[p, row, pl.ds(96, 32)],
                    format=plsc.PackFormat.INTERLEAVED,
                )
                j0, j1 = plsc.unpack(
                    je_v[p, row, pl.ds(64, 32)],
                    format=plsc.PackFormat.INTERLEAVED,
                )
                j2, j3 = plsc.unpack(
                    je_v[p, row, pl.ds(96, 32)],
                    format=plsc.PackFormat.INTERLEAVED,
                )
                p_ui = (u0 * i0 + u1 * i1) + (u2 * i2 + u3 * i3)
                p_uj = (u0 * j0 + u1 * j1) + (u2 * j2 + u3 * j3)
                pu[pl.ds(e * 17, 16)] = p_ui
                pj[pl.ds(e * 17, 16)] = p_uj
                sq = ((u0 * u0 + u1 * u1) + (u2 * u2 + u3 * u3)
                      + (i0 * i0 + i1 * i1) + (i2 * i2 + i3 * i3)
                      + (j0 * j0 + j1 * j1) + (j2 * j2 + j3 * j3))
                acc_sq = acc_sq + sq
            s_ui = zeros
            s_uj = zeros
            for c in range(16):
                col = lanes17 + c
                s_ui = s_ui + plsc.load_gather(pu, [col])
                s_uj = s_uj + plsc.load_gather(pj, [col])
            off = k * _CH + g * 16
            xv[pl.ds(off, 16)] = s_ui - s_uj
            bi = bi_v[pl.ds(off, 16)]
            bj = bj_v[pl.ds(off, 16)]
            return reg_acc + acc_sq + bi * bi + bj * bj
        return group

    reg_acc = zeros
    inflight = {0: fire(0)}
    for k in range(_NCH):
        if k + 1 < _NCH:
            inflight[k + 1] = fire(k + 1)
        for c in inflight.pop(k):
            c.wait()
        reg_acc = lax.fori_loop(0, _GPC, make_group(k % 2, k), reg_acc)
    regv[...] = reg_acc

    pltpu.sync_copy(xv, x_out.at[pl.ds(base, _BPW)])
    pltpu.sync_copy(regv, reg_out.at[pl.ds(wid * 16, 16)])


_sc_call = functools.partial(
    pl.kernel,
    out_type=[
        jax.ShapeDtypeStruct((_B,), jnp.float32),
        jax.ShapeDtypeStruct((_NW * 16,), jnp.float32),
    ],
    mesh=plsc.VectorSubcoreMesh(core_axis_name="c", subcore_axis_name="s"),
    compiler_params=pltpu.CompilerParams(
        needs_layout_passes=False, use_tc_tiling_on_sc=False
    ),
    scratch_types=[
        pltpu.VMEM((_NCH, _CH), jnp.int32),
        pltpu.VMEM((_NCH, _CH), jnp.int32),
        pltpu.VMEM((_NCH, _CH), jnp.int32),
        pltpu.VMEM((2, _CH, 2 * _D), jnp.bfloat16),
        pltpu.VMEM((2, _CH, 2 * _D), jnp.bfloat16),
        pltpu.VMEM((2, _CH, 2 * _D), jnp.bfloat16),
        pltpu.VMEM((_BPW,), jnp.float32),
        pltpu.VMEM((_BPW,), jnp.float32),
        pltpu.VMEM((_BPW,), jnp.float32),
        pltpu.VMEM((272,), jnp.float32),
        pltpu.VMEM((272,), jnp.float32),
        pltpu.VMEM((16,), jnp.float32),
        pltpu.SemaphoreType.DMA,
        pltpu.SemaphoreType.DMA,
        pltpu.SemaphoreType.DMA,
    ],
)(_sc_body)


_TBLK = 8192
_NROW = 100000


def _tp_body(ut_ref, vt_ref, o_ref):
    eye = jnp.eye(_D, dtype=jnp.float32)
    dn = (((0,), (0,)), ((), ()))
    tu = lax.dot_general(ut_ref[...], eye, dn,
                         preferred_element_type=jnp.float32)
    tv = lax.dot_general(vt_ref[...], eye, dn,
                         preferred_element_type=jnp.float32)
    o_ref[...] = jnp.concatenate([tu, tv], axis=1).astype(jnp.bfloat16)


_tp_build = pl.pallas_call(
    _tp_body,
    grid=((_NROW + _TBLK - 1) // _TBLK,),
    in_specs=[
        pl.BlockSpec((_D, _TBLK), lambda b: (0, b)),
        pl.BlockSpec((_D, _TBLK), lambda b: (0, b)),
    ],
    out_specs=pl.BlockSpec((_TBLK, 2 * _D), lambda b: (b, 0)),
    out_shape=jax.ShapeDtypeStruct((_NROW, 2 * _D), jnp.bfloat16),
)


def _tc_body(x_ref, reg_ref, o_ref):
    x = x_ref[...]
    neg = -x
    sp = jnp.maximum(neg, 0.0) + jnp.log(1.0 + jnp.exp(-jnp.abs(neg)))
    o_ref[0, 0] = jnp.sum(sp) + _WD * jnp.sum(reg_ref[...])


_tc_reduce = pl.pallas_call(
    _tc_body,
    out_shape=jax.ShapeDtypeStruct((1, 1), jnp.float32),
    out_specs=pl.BlockSpec(memory_space=pltpu.SMEM),
)


@jax.jit
def kernel(u, i, j, U, V, biasV):
    T = _tp_build(U.T, V.T)
    x, reg = _sc_call(u, i, j, T, biasV)
    out = _tc_reduce(x.reshape(_NW, _BPW), reg.reshape(_NW, 16))
    return out[0, 0]


# speedup vs baseline: 2.0410x; 2.0410x over previous
import functools

import jax
import jax.numpy as jnp
from jax import lax
from jax.experimental import pallas as pl
from jax.experimental.pallas import tpu as pltpu
from jax.experimental.pallas import tpu_sc as plsc

_WD = 0.01
_B = 16384
_D = 64
_NC = 2
_NS = 16
_NW = _NC * _NS
_BPW = _B // _NW
_CH = 128
_NCH = _BPW // _CH
_GPC = _CH // 16


def _sc_body(u_hbm, i_hbm, j_hbm, T_hbm, bias_hbm, x_out, reg_out,
             idx_u, idx_i, idx_j, ue_v, ie_v, je_v, bi_v, bj_v, xv, pu, pj,
             regv, sem_i, sem_a, sem_b):
    wid = lax.axis_index("s") * _NC + lax.axis_index("c")
    base = wid * _BPW

    stage = []
    for k in range(_NCH):
        src = pl.ds(base + k * _CH, _CH)
        stage.append(pltpu.async_copy(u_hbm.at[src], idx_u.at[k], sem_i))
        stage.append(pltpu.async_copy(i_hbm.at[src], idx_i.at[k], sem_i))
        stage.append(pltpu.async_copy(j_hbm.at[src], idx_j.at[k], sem_i))
    for c in stage:
        c.wait()

    sems = (sem_a, sem_b)

    def fire(k):
        p = k % 2
        sem = sems[p]
        sl = pl.ds(k * _CH, _CH)
        return [
            pltpu.async_copy(T_hbm.at[idx_u.at[k]], ue_v.at[p], sem),
            pltpu.async_copy(T_hbm.at[idx_i.at[k]], ie_v.at[p], sem),
            pltpu.async_copy(T_hbm.at[idx_j.at[k]], je_v.at[p], sem),
            pltpu.async_copy(bias_hbm.at[idx_i.at[k]], bi_v.at[sl], sem),
            pltpu.async_copy(bias_hbm.at[idx_j.at[k]], bj_v.at[sl], sem),
        ]

    lanes = lax.iota(jnp.int32, 16)
    lanes17 = lanes * 17
    zeros = jnp.zeros((16,), jnp.float32)

    def make_group(p, k):
        def group(g, reg_acc):
            acc_sq = zeros
            for e in range(16):
                row = g * 16 + e
                u0 = ue_v[p, row, pl.ds(0, 16)]
                u1 = ue_v[p, row, pl.ds(16, 16)]
                u2 = ue_v[p, row, pl.ds(32, 16)]
                u3 = ue_v[p, row, pl.ds(48, 16)]
                i0 = ie_v[p, row, pl.ds(64, 16)]
                i1 = ie_v[p, row, pl.ds(80, 16)]
                i2 = ie_v[p, row, pl.ds(96, 16)]
                i3 = ie_v[p, row, pl.ds(112, 16)]
                j0 = je_v[p, row, pl.ds(64, 16)]
                j1 = je_v[p, row, pl.ds(80, 16)]
                j2 = je_v[p, row, pl.ds(96, 16)]
                j3 = je_v[p, row, pl.ds(112, 16)]
                p_ui = (u0 * i0 + u1 * i1) + (u2 * i2 + u3 * i3)
                p_uj = (u0 * j0 + u1 * j1) + (u2 * j2 + u3 * j3)
                pu[pl.ds(e * 17, 16)] = p_ui
                pj[pl.ds(e * 17, 16)] = p_uj
                sq = ((u0 * u0 + u1 * u1) + (u2 * u2 + u3 * u3)
                      + (i0 * i0 + i1 * i1) + (i2 * i2 + i3 * i3)
                      + (j0 * j0 + j1 * j1) + (j2 * j2 + j3 * j3))
                acc_sq = acc_sq + sq
            s_ui = zeros
            s_uj = zeros
            for c in range(16):
                col = lanes17 + c
                s_ui = s_ui + plsc.load_gather(pu, [col])
                s_uj = s_uj + plsc.load_gather(pj, [col])
            off = k * _CH + g * 16
            xv[pl.ds(off, 16)] = s_ui - s_uj
            bi = bi_v[pl.ds(off, 16)]
            bj = bj_v[pl.ds(off, 16)]
            return reg_acc + acc_sq + bi * bi + bj * bj
        return group

    reg_acc = zeros
    inflight = {0: fire(0)}
    for k in range(_NCH):
        if k + 1 < _NCH:
            inflight[k + 1] = fire(k + 1)
        for c in inflight.pop(k):
            c.wait()
        reg_acc = lax.fori_loop(0, _GPC, make_group(k % 2, k), reg_acc)
    regv[...] = reg_acc

    pltpu.sync_copy(xv, x_out.at[pl.ds(base, _BPW)])
    pltpu.sync_copy(regv, reg_out.at[pl.ds(wid * 16, 16)])


_sc_call = functools.partial(
    pl.kernel,
    out_type=[
        jax.ShapeDtypeStruct((_B,), jnp.float32),
        jax.ShapeDtypeStruct((_NW * 16,), jnp.float32),
    ],
    mesh=plsc.VectorSubcoreMesh(core_axis_name="c", subcore_axis_name="s"),
    compiler_params=pltpu.CompilerParams(
        needs_layout_passes=False, use_tc_tiling_on_sc=False
    ),
    scratch_types=[
        pltpu.VMEM((_NCH, _CH), jnp.int32),
        pltpu.VMEM((_NCH, _CH), jnp.int32),
        pltpu.VMEM((_NCH, _CH), jnp.int32),
        pltpu.VMEM((2, _CH, 2 * _D), jnp.float32),
        pltpu.VMEM((2, _CH, 2 * _D), jnp.float32),
        pltpu.VMEM((2, _CH, 2 * _D), jnp.float32),
        pltpu.VMEM((_BPW,), jnp.float32),
        pltpu.VMEM((_BPW,), jnp.float32),
        pltpu.VMEM((_BPW,), jnp.float32),
        pltpu.VMEM((272,), jnp.float32),
        pltpu.VMEM((272,), jnp.float32),
        pltpu.VMEM((16,), jnp.float32),
        pltpu.SemaphoreType.DMA,
        pltpu.SemaphoreType.DMA,
        pltpu.SemaphoreType.DMA,
    ],
)(_sc_body)


_TBLK = 8192
_NROW = 100000


def _tp_body(ut_ref, vt_ref, o_ref):
    eye = jnp.eye(_D, dtype=jnp.float32)
    dn = (((0,), (0,)), ((), ()))
    tu = lax.dot_general(ut_ref[...], eye, dn,
                         preferred_element_type=jnp.float32)
    tv = lax.dot_general(vt_ref[...], eye, dn,
                         preferred_element_type=jnp.float32)
    o_ref[...] = jnp.concatenate([tu, tv], axis=1)


_tp_build = pl.pallas_call(
    _tp_body,
    grid=((_NROW + _TBLK - 1) // _TBLK,),
    in_specs=[
        pl.BlockSpec((_D, _TBLK), lambda b: (0, b)),
        pl.BlockSpec((_D, _TBLK), lambda b: (0, b)),
    ],
    out_specs=pl.BlockSpec((_TBLK, 2 * _D), lambda b: (b, 0)),
    out_shape=jax.ShapeDtypeStruct((_NROW, 2 * _D), jnp.float32),
)


def _tc_body(x_ref, reg_ref, o_ref):
    x = x_ref[...]
    neg = -x
    sp = jnp.maximum(neg, 0.0) + jnp.log(1.0 + jnp.exp(-jnp.abs(neg)))
    o_ref[0, 0] = jnp.sum(sp) + _WD * jnp.sum(reg_ref[...])


_tc_reduce = pl.pallas_call(
    _tc_body,
    out_shape=jax.ShapeDtypeStruct((1, 1), jnp.float32),
    out_specs=pl.BlockSpec(memory_space=pltpu.SMEM),
)


@jax.jit
def kernel(u, i, j, U, V, biasV):
    T = _tp_build(U.T, V.T)
    x, reg = _sc_call(u, i, j, T, biasV)
    out = _tc_reduce(x, reg)
    return out[0, 0]
